# Initial kernel scaffold; baseline (speedup 1.0000x reference)
import jax
import jax.numpy as jnp
from jax.experimental import pallas as pl


def kernel(img_feat_map, bev_feat_map, top_anchors, ground_plane, img_mask, bev_mask, W1, b1, Wc, bc, Wo, bo, Wa, ba):
    raise NotImplementedError("write your pallas kernel here")


# speedup vs baseline: 28.4302x; 28.4302x over previous
import functools

import jax
import jax.numpy as jnp
from jax import lax
from jax.experimental import pallas as pl
from jax.experimental.pallas import tpu as pltpu
from jax.experimental.pallas import tpu_sc as plsc

ROI_SIZE = 7
NMS_THRESH = 0.01
NMS_SIZE = 100
X_MIN, X_MAX = -40.0, 40.0
Z_MIN, Z_MAX = 0.0, 70.0
F_CAM, CU, CV = 180.0, 150.0, 45.0

NP_ROIS = 5120
R_PER_W = 160
BLK = 512
GRID_R, GRID_C = 8, NP_ROIS // 8
IMG_H, IMG_W = 90, 300
BEV_H, BEV_W = 175, 200
BEV_BASE = IMG_H * IMG_W
TAB_ROWS = BEV_BASE + BEV_H * BEV_W


def _prep_body(at_ref, msk_ref, idx_ref, w_ref):
    i = pl.program_id(0)
    f32 = jnp.float32
    cx = at_ref[0]
    cy = at_ref[1]
    cz = at_ref[2]
    dx = at_ref[3]
    dy = at_ref[4]
    dz = at_ref[5]
    mi = msk_ref[0, 0]
    mb = msk_ref[0, 1]
    hi = mi / (mi + mb)
    hb = mb / (mi + mb)

    bx1 = jnp.clip((cx - dx / 2.0 - X_MIN) / (X_MAX - X_MIN) * float(BEV_W), 0.0, BEV_W - 1.0)
    bx2 = jnp.clip((cx + dx / 2.0 - X_MIN) / (X_MAX - X_MIN) * float(BEV_W), 0.0, BEV_W - 1.0)
    by1 = jnp.clip((cz - dz / 2.0 - Z_MIN) / (Z_MAX - Z_MIN) * float(BEV_H), 0.0, BEV_H - 1.0)
    by2 = jnp.clip((cz + dz / 2.0 - Z_MIN) / (Z_MAX - Z_MIN) * float(BEV_H), 0.0, BEV_H - 1.0)

    zc = jnp.maximum(cz - dz / 2.0, 1.0)
    u1 = F_CAM * (cx - dx / 2.0) / zc + CU
    u2 = F_CAM * (cx + dx / 2.0) / zc + CU
    v1 = F_CAM * (cy - dy / 2.0) / zc + CV
    v2 = F_CAM * (cy + dy / 2.0) / zc + CV
    ix1 = jnp.clip(jnp.minimum(u1, u2), 0.0, IMG_W - 1.0)
    ix2 = jnp.clip(jnp.maximum(u1, u2), 0.0, IMG_W - 1.0)
    iy1 = jnp.clip(jnp.minimum(v1, v2), 0.0, IMG_H - 1.0)
    iy2 = jnp.clip(jnp.maximum(v1, v2), 0.0, IMG_H - 1.0)

    ifl = i.astype(f32) + 0.5

    def taps(x1, y1, x2, y2, jc, W, H, wscale, rowbase):
        bw = (x2 - x1) / float(ROI_SIZE)
        bh = (y2 - y1) / float(ROI_SIZE)
        gx = x1 + jc * bw - 0.5
        gy = y1 + ifl * bh - 0.5
        x0 = jnp.floor(gx)
        y0 = jnp.floor(gy)
        wx = gx - x0
        wy = gy - y0
        x0i = jnp.clip(x0.astype(jnp.int32), 0, W - 1)
        x1i = jnp.clip(x0i + 1, 0, W - 1)
        y0i = jnp.clip(y0.astype(jnp.int32), 0, H - 1)
        y1i = jnp.clip(y0i + 1, 0, H - 1)
        rows = [rowbase + y0i * W + x0i, rowbase + y0i * W + x1i,
                rowbase + y1i * W + x0i, rowbase + y1i * W + x1i]
        ws = [((1.0 - wy) * (1.0 - wx)) * wscale, ((1.0 - wy) * wx) * wscale,
              (wy * (1.0 - wx)) * wscale, (wy * wx) * wscale]
        return rows, ws

    for j in range(ROI_SIZE):
        jc = jnp.float32(j + 0.5)
        irows, iws = taps(ix1, iy1, ix2, iy2, jc, IMG_W, IMG_H, hi, 0)
        brows, bws = taps(bx1, by1, bx2, by2, jc, BEV_W, BEV_H, hb, BEV_BASE)
        for k in range(4):
            idx_ref[0, j * 8 + k] = irows[k]
            w_ref[0, j * 8 + k] = iws[k]
            idx_ref[0, j * 8 + 4 + k] = brows[k]
            w_ref[0, j * 8 + 4 + k] = bws[k]


def _run_prep(aT, masks):
    grid = (ROI_SIZE,)
    return pl.pallas_call(
        _prep_body,
        grid=grid,
        in_specs=[
            pl.BlockSpec((6, GRID_R, GRID_C), lambda i: (0, 0, 0)),
            pl.BlockSpec((1, 2), lambda i: (0, 0)),
        ],
        out_specs=[
            pl.BlockSpec((1, 56, GRID_R, GRID_C), lambda i: (i, 0, 0, 0)),
            pl.BlockSpec((1, 56, GRID_R, GRID_C), lambda i: (i, 0, 0, 0)),
        ],
        out_shape=[
            jax.ShapeDtypeStruct((ROI_SIZE, 56, GRID_R, GRID_C), jnp.int32),
            jax.ShapeDtypeStruct((ROI_SIZE, 56, GRID_R, GRID_C), jnp.float32),
        ],
    )(aT, masks)


_NC = 2
_NS = 16


def _sc_gather(table, idxT, wT):
    mesh = plsc.VectorSubcoreMesh(core_axis_name="c", subcore_axis_name="s")

    @functools.partial(
        pl.kernel,
        mesh=mesh,
        out_type=jax.ShapeDtypeStruct((NP_ROIS, 1568), jnp.float32),
        compiler_params=pltpu.CompilerParams(use_tc_tiling_on_sc=False,
                                             needs_layout_passes=False),
        scratch_types=[
            pltpu.VMEM((2, 1, 400), jnp.int32),
            pltpu.VMEM((2, 1, 400), jnp.float32),
            pltpu.VMEM((2, 392, 32), jnp.float32),
            pltpu.VMEM((2, 1, 1568), jnp.float32),
            pltpu.SemaphoreType.DMA,
            pltpu.SemaphoreType.DMA,
            pltpu.SemaphoreType.DMA,
            pltpu.SemaphoreType.DMA,
            pltpu.SemaphoreType.DMA,
            pltpu.SemaphoreType.DMA,
        ],
    )
    def k(tab_hbm, idx_hbm, w_hbm, feat_hbm, idx_v, w_v, rows_v, out_v,
          sem_i0, sem_i1, sem_g0, sem_g1, sem_w0, sem_w1):
        wid = lax.axis_index("s") * _NC + lax.axis_index("c")
        base = wid * R_PER_W
        sem_i = [sem_i0, sem_i1]
        sem_g = [sem_g0, sem_g1]
        sem_w = [sem_w0, sem_w1]
        iota49 = lax.broadcasted_iota(jnp.int32, (16,), 0) * 49
        lane_k = [jnp.full((16,), kk, jnp.int32) for kk in range(8)]

        def idx_copy(r, slot, do_wait):
            a = pltpu.make_async_copy(
                idx_hbm.at[pl.ds(base + r, 1), :], idx_v.at[slot], sem_i[slot])
            b = pltpu.make_async_copy(
                w_hbm.at[pl.ds(base + r, 1), :], w_v.at[slot], sem_i[slot])
            if do_wait:
                a.wait()
                b.wait()
            else:
                a.start()
                b.start()

        def gathers(slot, do_wait):
            for k7 in range(7):
                c = pltpu.make_async_copy(
                    tab_hbm.at[idx_v.at[slot, 0, pl.ds(56 * k7, 56)]],
                    rows_v.at[slot, pl.ds(56 * k7, 56)],
                    sem_g[slot])
                if do_wait:
                    c.wait()
                else:
                    c.start()

        def out_write(r, slot, do_wait):
            c = pltpu.make_async_copy(
                out_v.at[slot],
                feat_hbm.at[pl.ds(base + r, 1), :],
                sem_w[slot])
            if do_wait:
                c.wait()
            else:
                c.start()

        def blend(slot):
            def body(p, carry):
                pk = p * 8
                wvec = w_v[slot, 0, pl.ds(pk, 16)]
                wb = [wvec.at[lane_k[kk]].get(mode="promise_in_bounds")
                      for kk in range(8)]
                for half in range(2):
                    hs = pl.ds(half * 16, 16)
                    vi = rows_v[slot, pk, hs] * wb[0]
                    for k4 in range(1, 4):
                        vi = vi + rows_v[slot, pk + k4, hs] * wb[k4]
                    vb = rows_v[slot, pk + 4, hs] * wb[4]
                    for k4 in range(5, 8):
                        vb = vb + rows_v[slot, pk + k4, hs] * wb[k4]
                    plsc.store_scatter(out_v.at[slot, 0],
                                       [iota49 + (half * 16 * 49) + p],
                                       vi + vb)
                return carry
            lax.fori_loop(0, 49, body, 0, unroll=7)

        def outer(r, carry):
            idx_copy(r, 0, False)
            idx_copy(r, 0, True)
            gathers(0, False)
            gathers(0, True)
            blend(0)
            out_write(r, 0, False)
            out_write(r, 0, True)
            return carry

        lax.fori_loop(0, R_PER_W, outer, 0)

    return k(table, idxT, wT)


def _mlp_body(feat_ref, w1_ref, b1_ref, wh_ref, bh_ref, s_ref):
    h = jnp.dot(feat_ref[...], w1_ref[...], preferred_element_type=jnp.float32)
    h = jnp.maximum(h + b1_ref[...], 0.0)
    s = jnp.dot(h, wh_ref[...], preferred_element_type=jnp.float32)
    s_ref[...] = s + bh_ref[...]


def _run_mlp(feat_p, W1, b1, Wh, bh):
    grid = (NP_ROIS // BLK,)
    return pl.pallas_call(
        _mlp_body,
        grid=grid,
        in_specs=[
            pl.BlockSpec((BLK, 1568), lambda i: (i, 0)),
            pl.BlockSpec((1568, 256), lambda i: (0, 0)),
            pl.BlockSpec((1, 256), lambda i: (0, 0)),
            pl.BlockSpec((256, 128), lambda i: (0, 0)),
            pl.BlockSpec((1, 128), lambda i: (0, 0)),
        ],
        out_specs=pl.BlockSpec((BLK, 128), lambda i: (i, 0)),
        out_shape=jax.ShapeDtypeStruct((NP_ROIS, 128), jnp.float32),
    )(feat_p, W1, b1, Wh, bh)


def _nms_body(st_ref, at_ref, gp_ref, comb_ref):
    f32 = jnp.float32
    obj0 = st_ref[0]
    obj1 = st_ref[1]
    offs = [st_ref[2 + k] for k in range(10)]
    ang0 = st_ref[12]
    ang1 = st_ref[13]
    anc = [at_ref[k] for k in range(6)]
    g0 = gp_ref[0, 0]
    g1 = gp_ref[0, 1]
    g2 = gp_ref[0, 2]
    g3 = gp_ref[0, 3]

    pa = [anc[k] + offs[k] for k in range(6)]
    gyg = -(g0 * pa[0] + g2 * pa[2] + g3) / g1
    orient = jnp.arctan2(ang1, ang0)
    mx = jnp.maximum(obj0, obj1)
    e0 = jnp.exp(obj0 - mx)
    e1 = jnp.exp(obj1 - mx)
    esum = e0 + e1
    soft0 = e0 / esum
    soft1 = e1 / esum

    bx1 = pa[0] - pa[3] / 2.0
    by1 = pa[2] - pa[5] / 2.0
    bx2 = pa[0] + pa[3] / 2.0
    by2 = pa[2] + pa[5] / 2.0
    areas = (bx2 - bx1) * (by2 - by1)
    score = obj1

    ridx = jax.lax.broadcasted_iota(jnp.int32, (GRID_R, GRID_C), 0)
    cidx = jax.lax.broadcasted_iota(jnp.int32, (GRID_R, GRID_C), 1)
    idxg = ridx * GRID_C + cidx
    valid = idxg < 5000

    NEG = jnp.float32(-3.0e38)
    BIGI = jnp.int32(2 ** 30)

    gsc = jnp.where(valid, score, NEG)
    gm = jnp.max(gsc)
    gsel = jnp.min(jnp.where((gsc == gm) & valid, idxg, BIGI))

    lanes = jax.lax.broadcasted_iota(jnp.int32, (1, 128), 1)

    extract_grids = [soft0, soft1] + pa + offs + [gyg, orient, areas, bx1, by1, bx2, by2]

    def step(t, avail_i):
        avail = avail_i != 0
        s_av = jnp.where(avail, score, NEG)
        m = jnp.max(s_av)
        any_avail = m > NEG
        sel_cand = jnp.min(jnp.where((s_av == m) & avail, idxg, BIGI))
        sel = jnp.where(any_avail, sel_cand, gsel)
        eq = idxg == sel
        eqf = eq.astype(f32)
        vals = [jnp.sum(g * eqf) for g in extract_grids]
        ars = vals[20]
        x1s, y1s, x2s, y2s = vals[21], vals[22], vals[23], vals[24]
        xx1 = jnp.maximum(x1s, bx1)
        yy1 = jnp.maximum(y1s, by1)
        xx2 = jnp.minimum(x2s, bx2)
        yy2 = jnp.minimum(y2s, by2)
        inter = jnp.maximum(xx2 - xx1, 0.0) * jnp.maximum(yy2 - yy1, 0.0)
        ious = inter / (ars + areas - inter + 1e-8)
        avail_i = jnp.where((ious > NMS_THRESH) | eq, 0, avail_i)
        row = jnp.zeros((1, 128), f32)
        for k in range(20):
            row = jnp.where(lanes == k, vals[k], row)
        comb_ref[pl.ds(t, 1), :] = row
        return avail_i

    jax.lax.fori_loop(0, NMS_SIZE, step, valid.astype(jnp.int32))


def _run_nms(sT, aT, gp):
    return pl.pallas_call(
        _nms_body,
        in_specs=[
            pl.BlockSpec((16, GRID_R, GRID_C), lambda: (0, 0, 0)),
            pl.BlockSpec((6, GRID_R, GRID_C), lambda: (0, 0, 0)),
            pl.BlockSpec((1, 4), lambda: (0, 0)),
        ],
        out_specs=pl.BlockSpec((NMS_SIZE, 128), lambda: (0, 0)),
        out_shape=jax.ShapeDtypeStruct((NMS_SIZE, 128), jnp.float32),
    )(sT, aT, gp)


def kernel(img_feat_map, bev_feat_map, top_anchors, ground_plane, img_mask, bev_mask, W1, b1, Wc, bc, Wo, bo, Wa, ba):
    N = top_anchors.shape[0]

    timg = jnp.transpose(img_feat_map[0], (1, 2, 0)).reshape(BEV_BASE, 32)
    tbev = jnp.transpose(bev_feat_map[0], (1, 2, 0)).reshape(BEV_H * BEV_W, 32)
    table = jnp.concatenate([timg, tbev], axis=0)

    anchors_p = jnp.pad(top_anchors, ((0, NP_ROIS - N), (0, 0)))
    aT = jnp.transpose(anchors_p).reshape(6, GRID_R, GRID_C)
    masks = jnp.stack([img_mask[0], bev_mask[0]]).reshape(1, 2)

    idx4, w4 = _run_prep(aT, masks)
    idxT = jnp.pad(jnp.transpose(idx4.reshape(392, NP_ROIS)), ((0, 0), (0, 8)))
    wT = jnp.pad(jnp.transpose(w4.reshape(392, NP_ROIS)), ((0, 0), (0, 8)))

    feat_p = _sc_gather(table, idxT, wT)

    Wh = jnp.zeros((256, 128), jnp.float32)
    Wh = Wh.at[:, 0:2].set(Wc).at[:, 2:12].set(Wo).at[:, 12:14].set(Wa)
    bh = jnp.zeros((128,), jnp.float32)
    bh = bh.at[0:2].set(bc).at[2:12].set(bo).at[12:14].set(ba)

    s_all = _run_mlp(feat_p, W1, b1.reshape(1, 256), Wh, bh.reshape(1, 128))

    sT = jnp.transpose(s_all[:, :16]).reshape(16, GRID_R, GRID_C)
    gp = ground_plane.reshape(1, 4)

    comb = _run_nms(sT, aT, gp)

    top_scores_soft = comb[:NMS_SIZE, 0:2]
    top_pred_anchors = comb[:NMS_SIZE, 2:8]
    p4c = comb[:NMS_SIZE, 8:18]
    pa0 = comb[:NMS_SIZE, 2:3]
    gyc = comb[:NMS_SIZE, 18:19]
    pa25 = comb[:NMS_SIZE, 4:8]
    orient = comb[:NMS_SIZE, 19]
    predictions_box = jnp.concatenate([pa0, gyc, pa25, orient[:, None]], axis=1)
    return (top_scores_soft, (top_pred_anchors, p4c, predictions_box), orient)
